# baseline (speedup 1.0000x reference)
import jax
from jax import lax
import jax.numpy as jnp
from jax.experimental import pallas as pl
from jax.experimental.pallas import tpu as pltpu
from jax.experimental.pallas import tpu_sc as plsc

DIM = 64
PAD = 128
NW = 32
NS = 16
W = 320


def _gather_one(table, idx_flat, n):
    rwl = n // NW
    C = rwl // W
    W2 = W // 2

    mesh = plsc.VectorSubcoreMesh(core_axis_name="c", subcore_axis_name="s")

    @pl.kernel(
        out_type=jax.ShapeDtypeStruct((n // 2, PAD), jnp.float32),
        mesh=mesh,
        compiler_params=pltpu.CompilerParams(use_tc_tiling_on_sc=False),
        scratch_types=[
            pltpu.VMEM((rwl,), jnp.int32),
            pltpu.VMEM((W, DIM), jnp.float32),
            pltpu.VMEM((W, DIM), jnp.float32),
            pltpu.VMEM((W2, PAD), jnp.float32),
            pltpu.VMEM((W2, PAD), jnp.float32),
            pltpu.SemaphoreType.DMA,
            pltpu.SemaphoreType.DMA,
            pltpu.SemaphoreType.DMA,
            pltpu.SemaphoreType.DMA,
        ],
    )
    def gather_kernel(tab_hbm, idx_hbm, out_hbm,
                      idx_v, g0, g1, o0, o1, gs0, gs1, ws0, ws1):
        wid = lax.axis_index("c") * NS + lax.axis_index("s")
        ibase = wid * rwl
        obase = wid * (rwl // 2)
        pltpu.sync_copy(idx_hbm.at[pl.ds(ibase, rwl)], idx_v)

        def gsrc(c):
            return tab_hbm.at[idx_v.at[pl.ds(c * W, W)]]

        def pack(gbuf, obuf):
            @pl.loop(0, W2)
            def _(r):
                for h in range(2):
                    for k in range(DIM // 16):
                        obuf[r, pl.ds(h * DIM + k * 16, 16)] = (
                            gbuf[2 * r + h, pl.ds(k * 16, 16)])

        pltpu.async_copy(gsrc(0), g0, gs0)
        pltpu.async_copy(gsrc(1), g1, gs1)

        def stage(c, gbuf, gsem, obuf, wsem):
            pltpu.make_async_copy(gsrc(c), gbuf, gsem).wait()

            @pl.when(c >= 2)
            def _():
                pltpu.make_async_copy(
                    obuf, out_hbm.at[pl.ds(obase + (c - 2) * W2, W2)],
                    wsem).wait()

            pack(gbuf, obuf)
            pltpu.async_copy(
                obuf, out_hbm.at[pl.ds(obase + c * W2, W2)], wsem)

            @pl.when(c + 2 < C)
            def _():
                pltpu.async_copy(gsrc(c + 2), gbuf, gsem)

        @pl.loop(0, C, step=2)
        def _(c):
            stage(c, g0, gs0, o0, ws0)
            stage(c + 1, g1, gs1, o1, ws1)

        pltpu.make_async_copy(
            o0, out_hbm.at[pl.ds(obase + (C - 2) * W2, W2)], ws0).wait()
        pltpu.make_async_copy(
            o1, out_hbm.at[pl.ds(obase + (C - 1) * W2, W2)], ws1).wait()

    return gather_kernel(table, idx_flat)


def kernel(head, tail, table):
    B, L = head.shape
    n = B * L
    head_i = head.reshape(n).astype(jnp.int32)
    tail_i = tail.reshape(n).astype(jnp.int32)
    ho2 = _gather_one(table, head_i, n)
    to2 = _gather_one(table, tail_i, n)
    return ho2.reshape(B, L, DIM), to2.reshape(B, L, DIM)


# speedup vs baseline: 1.4502x; 1.4502x over previous
import jax
from jax import lax
import jax.numpy as jnp
from jax.experimental import pallas as pl
from jax.experimental.pallas import tpu as pltpu
from jax.experimental.pallas import tpu_sc as plsc

DIM = 64
PAD = 128
NW = 32
NS = 16
CB = 8
NB = 8192


def _transpose_pad(table):
    V = table.shape[0]
    tab_t = jnp.transpose(table)

    def body(t_ref, o_ref):
        o_ref[:, :DIM] = jnp.transpose(t_ref[...])

    return pl.pallas_call(
        body,
        grid=(pl.cdiv(V, NB),),
        in_specs=[pl.BlockSpec((DIM, NB), lambda i: (0, i))],
        out_specs=pl.BlockSpec((NB, PAD), lambda i: (i, 0)),
        out_shape=jax.ShapeDtypeStruct((V, PAD), jnp.float32),
    )(tab_t)


def _gather_one(tab128, idx_flat, B, L):
    rw = B // NW
    rwl = rw * L
    cbl = CB * L
    C = rw // CB

    mesh = plsc.VectorSubcoreMesh(core_axis_name="c", subcore_axis_name="s")

    @pl.kernel(
        out_type=jax.ShapeDtypeStruct((B, L, DIM), jnp.float32),
        mesh=mesh,
        scratch_types=[
            pltpu.VMEM((rwl,), jnp.int32),
            pltpu.VMEM((cbl, PAD), jnp.float32),
            pltpu.VMEM((cbl, PAD), jnp.float32),
            pltpu.VMEM((CB, L, DIM), jnp.float32),
            pltpu.VMEM((CB, L, DIM), jnp.float32),
            pltpu.SemaphoreType.DMA,
            pltpu.SemaphoreType.DMA,
            pltpu.SemaphoreType.DMA,
            pltpu.SemaphoreType.DMA,
        ],
    )
    def gather_kernel(tab_hbm, idx_hbm, out_hbm,
                      idx_v, g0, g1, o0, o1, gs0, gs1, ws0, ws1):
        wid = lax.axis_index("c") * NS + lax.axis_index("s")
        ibase = wid * rwl
        obase = wid * rw
        pltpu.sync_copy(idx_hbm.at[pl.ds(ibase, rwl)], idx_v)

        def gsrc(c):
            return tab_hbm.at[idx_v.at[pl.ds(c * cbl, cbl)]]

        def compact(gbuf, obuf):
            @pl.loop(0, CB)
            def _(b):
                @pl.loop(0, L)
                def _(j):
                    r = b * L + j
                    for k in range(DIM // 16):
                        obuf[b, j, pl.ds(k * 16, 16)] = (
                            gbuf[r, pl.ds(k * 16, 16)])

        pltpu.async_copy(gsrc(0), g0, gs0)
        pltpu.async_copy(gsrc(1), g1, gs1)

        def stage(c, gbuf, gsem, obuf, wsem):
            pltpu.make_async_copy(gsrc(c), gbuf, gsem).wait()

            @pl.when(c >= 2)
            def _():
                pltpu.make_async_copy(
                    obuf, out_hbm.at[pl.ds(obase + (c - 2) * CB, CB)],
                    wsem).wait()

            compact(gbuf, obuf)
            pltpu.async_copy(
                obuf, out_hbm.at[pl.ds(obase + c * CB, CB)], wsem)

            @pl.when(c + 2 < C)
            def _():
                pltpu.async_copy(gsrc(c + 2), gbuf, gsem)

        @pl.loop(0, C, step=2)
        def _(c):
            stage(c, g0, gs0, o0, ws0)
            stage(c + 1, g1, gs1, o1, ws1)

        pltpu.make_async_copy(
            o0, out_hbm.at[pl.ds(obase + (C - 2) * CB, CB)], ws0).wait()
        pltpu.make_async_copy(
            o1, out_hbm.at[pl.ds(obase + (C - 1) * CB, CB)], ws1).wait()

    return gather_kernel(tab128, idx_flat)


def kernel(head, tail, table):
    B, L = head.shape
    n = B * L
    head_i = head.reshape(n).astype(jnp.int32)
    tail_i = tail.reshape(n).astype(jnp.int32)
    tab128 = _transpose_pad(table)
    ho = _gather_one(tab128, head_i, B, L)
    to = _gather_one(tab128, tail_i, B, L)
    return ho, to
